# baseline (speedup 1.0000x reference)
import functools

import jax
import jax.numpy as jnp
from jax import lax
from jax.experimental import pallas as pl
from jax.experimental.pallas import tpu as pltpu
from jax.experimental.pallas import tpu_sc as plsc

_NUM_CORES = 2
_NUM_SUBCORES = 16
_NUM_WORKERS = _NUM_CORES * _NUM_SUBCORES


@functools.partial(jax.jit, static_argnums=(0, 1, 2))
def _pos_embed_sc(batch, seq_len, emb_dim, w_pos):
    rows_per_w = seq_len // _NUM_WORKERS

    mesh = plsc.VectorSubcoreMesh(
        core_axis_name="c", subcore_axis_name="s",
        num_cores=_NUM_CORES, num_subcores=_NUM_SUBCORES,
    )

    @functools.partial(
        pl.kernel,
        mesh=mesh,
        out_type=jax.ShapeDtypeStruct((batch * seq_len, emb_dim), jnp.float32),
        scratch_types=[
            pltpu.SemaphoreType.DMA,
        ],
    )
    def k(w_hbm, out_hbm, sem):
        wid = lax.axis_index("s") * _NUM_CORES + lax.axis_index("c")
        base = wid * rows_per_w
        src = w_hbm.at[pl.ds(base, rows_per_w)]
        copies = [
            pltpu.async_copy(
                src, out_hbm.at[pl.ds(b * seq_len + base, rows_per_w)], sem)
            for b in range(batch)
        ]
        for cp in copies:
            cp.wait()

    return k(w_pos)


def kernel(tokens, W_pos):
    batch, seq_len = tokens.shape
    emb_dim = W_pos.shape[1]
    out = _pos_embed_sc(batch, seq_len, emb_dim, W_pos)
    return out.reshape(batch, seq_len, emb_dim)


# speedup vs baseline: 44.5732x; 44.5732x over previous
import functools

import jax
import jax.numpy as jnp
from jax import lax
from jax.experimental import pallas as pl
from jax.experimental.pallas import tpu as pltpu
from jax.experimental.pallas import tpu_sc as plsc

_NUM_CORES = 2
_NUM_SUBCORES = 16
_NUM_WORKERS = _NUM_CORES * _NUM_SUBCORES


@functools.partial(jax.jit, static_argnums=(0, 1, 2))
def _pos_embed_sc(batch, seq_len, emb_dim, w_pos):
    rows_per_w = seq_len // _NUM_WORKERS
    chunk = min(rows_per_w, 32)
    n_chunks = rows_per_w // chunk
    nbuf = min(n_chunks, 3)

    mesh = plsc.VectorSubcoreMesh(
        core_axis_name="c", subcore_axis_name="s",
        num_cores=_NUM_CORES, num_subcores=_NUM_SUBCORES,
    )

    @functools.partial(
        pl.kernel,
        mesh=mesh,
        out_type=jax.ShapeDtypeStruct((batch * seq_len, emb_dim), jnp.float32),
        scratch_types=[
            [pltpu.VMEM((chunk, emb_dim), jnp.float32) for _ in range(nbuf)],
            [pltpu.SemaphoreType.DMA for _ in range(nbuf)],
            [pltpu.SemaphoreType.DMA for _ in range(nbuf)],
        ],
    )
    def k(w_hbm, out_hbm, bufs, gsems, ssems):
        wid = lax.axis_index("s") * _NUM_CORES + lax.axis_index("c")
        base = wid * rows_per_w

        def gather(c):
            i = c % nbuf
            return pltpu.async_copy(
                w_hbm.at[pl.ds(base + c * chunk, chunk)], bufs[i], gsems[i])

        def scatter(c):
            i = c % nbuf
            row0 = base + c * chunk
            return [
                pltpu.async_copy(
                    bufs[i], out_hbm.at[pl.ds(b * seq_len + row0, chunk)],
                    ssems[i])
                for b in range(batch)
            ]

        g = {c: gather(c) for c in range(min(nbuf - 1, n_chunks))}
        s = {}
        for c in range(n_chunks):
            g.pop(c).wait()
            s[c] = scatter(c)
            nxt = c + nbuf - 1
            if nxt < n_chunks:
                prev = nxt - nbuf
                if prev >= 0:
                    for h in s.pop(prev):
                        h.wait()
                g[nxt] = gather(nxt)
        for c in sorted(s):
            for h in s[c]:
                h.wait()

    return k(w_pos)


def kernel(tokens, W_pos):
    batch, seq_len = tokens.shape
    emb_dim = W_pos.shape[1]
    out = _pos_embed_sc(batch, seq_len, emb_dim, W_pos)
    return out.reshape(batch, seq_len, emb_dim)
